# baseline (speedup 1.0000x reference)
import functools
import math

import jax
import jax.numpy as jnp
from jax import lax
from jax.experimental import pallas as pl
from jax.experimental.pallas import tpu as pltpu
from jax.experimental.pallas import tpu_sc as plsc


def _sc_gather(ids_flat, table, c):
    T = ids_flat.shape[0]
    V, Hd = table.shape
    NC, NS = 2, 16
    NW = NC * NS
    t_per_w = T // NW
    CH = 128
    n_ch = t_per_w // CH

    mesh = plsc.VectorSubcoreMesh(core_axis_name="c", subcore_axis_name="s")

    @functools.partial(
        pl.kernel,
        out_type=(
            jax.ShapeDtypeStruct((T, Hd), jnp.float32),
            jax.ShapeDtypeStruct((T,), jnp.float32),
        ),
        mesh=mesh,
        scratch_types=[
            pltpu.VMEM((t_per_w,), jnp.int32),
            pltpu.VMEM((CH, Hd), jnp.float32),
            pltpu.VMEM((CH, Hd), jnp.float32),
            pltpu.VMEM((CH, Hd), jnp.float32),
            pltpu.VMEM((CH, Hd), jnp.float32),
            pltpu.VMEM((t_per_w,), jnp.float32),
            pltpu.SemaphoreType.DMA,
            pltpu.SemaphoreType.DMA,
            pltpu.SemaphoreType.DMA,
        ],
    )
    def sc_kernel(ids_hbm, table_hbm, c_hbm, outE_hbm, outc_hbm,
                  idx_v, b0, b1, b2, b3, cbuf, semG, semW, semC):
        wid = lax.axis_index("s") * NC + lax.axis_index("c")
        base = wid * t_per_w
        pltpu.sync_copy(ids_hbm.at[pl.ds(base, t_per_w)], idx_v)

        ccp = [pltpu.async_copy(c_hbm.at[idx_v.at[pl.ds(i * CH, CH)]],
                                cbuf.at[pl.ds(i * CH, CH)], semC)
               for i in range(n_ch)]

        bufs = (b0, b1, b2, b3)

        def gather(i):
            return pltpu.async_copy(table_hbm.at[idx_v.at[pl.ds(i * CH, CH)]],
                                    bufs[i % 4], semG)

        gcp = [gather(0), gather(1)]
        wcp = []
        for i in range(n_ch):
            if i >= 2:
                wcp[i - 2].wait()
            if i + 2 < n_ch:
                gcp.append(gather(i + 2))
            gcp[i].wait()
            wcp.append(pltpu.async_copy(
                bufs[i % 4], outE_hbm.at[pl.ds(base + i * CH, CH)], semW))
        wcp[n_ch - 2].wait()
        wcp[n_ch - 1].wait()
        for cp in ccp:
            cp.wait()
        pltpu.sync_copy(cbuf, outc_hbm.at[pl.ds(base, t_per_w)])

    return sc_kernel(ids_flat, table, c)


def _tc_body(e_ref, c_ref, w_ref, b_ref, m_ref, loss_ref, mask_ref,
             pre, nsub, rs):
    x = e_ref[...]
    xb = (x * pre).astype(jnp.bfloat16)
    m01 = m_ref[...]
    w2 = w_ref[...]
    ones = jnp.ones((1, rs), dtype=jnp.bfloat16)
    bias = b_ref[0]
    parts = []
    for s in range(nsub):
        xs = xb[s * rs:(s + 1) * rs, :]
        lhs = jnp.concatenate([xs, w2], axis=0)
        gp = lax.dot_general(lhs, xs, (((1,), (1,)), ((), ())),
                             preferred_element_type=jnp.float32)
        g = gp[:rs, :]
        p = gp[rs:rs + 1, :]
        em = jnp.exp2(g) * m01
        em_b = em.astype(jnp.bfloat16)
        denom = jnp.sum(em, axis=0, keepdims=True)
        pp = (p / denom).astype(jnp.bfloat16)
        z = jnp.dot(pp, em_b, preferred_element_type=jnp.float32) + bias
        lq = -jnp.log(1.0 + jnp.exp(-z))
        cb = c_ref[0, s, :].reshape(1, rs)
        parts.append(jnp.sum(cb * (jnp.log(cb) - lq)))
        mask_ref[0, s, :] = jnp.where(cb >= 0.5, 1.0, 0.0).reshape(rs)
    @pl.when(pl.program_id(0) == 0)
    def _():
        loss_ref[0, 0] = 0.0
    loss_ref[0, 0] += sum(parts)


def _tc_attention(E_flat, c_flat, W, bias, Lp, PB=8, NSUB=32):
    T, Hd = E_flat.shape
    RS = PB * Lp
    R = RS * NSUB
    G = T // R
    c3 = c_flat.reshape(G, NSUB, RS)
    blk = jnp.arange(RS, dtype=jnp.int32) // Lp
    mask01 = (blk[:, None] == blk[None, :]).astype(jnp.float32)
    pre = math.sqrt(math.log2(math.e) / math.sqrt(Hd))
    w2 = jnp.zeros((8, Hd), jnp.float32).at[0].set(W.reshape(Hd) / pre)
    w2 = w2.astype(jnp.bfloat16)
    grid = (G,)
    loss_sum, mask = pl.pallas_call(
        functools.partial(_tc_body, pre=pre, nsub=NSUB, rs=RS),
        grid=grid,
        in_specs=[
            pl.BlockSpec((R, Hd), lambda i: (i, 0)),
            pl.BlockSpec((1, NSUB, RS), lambda i: (i, 0, 0)),
            pl.BlockSpec((8, Hd), lambda i: (0, 0)),
            pl.BlockSpec(memory_space=pltpu.SMEM),
            pl.BlockSpec((RS, RS), lambda i: (0, 0)),
        ],
        out_specs=[
            pl.BlockSpec((1, 1), lambda i: (0, 0), memory_space=pltpu.SMEM),
            pl.BlockSpec((1, NSUB, RS), lambda i: (i, 0, 0)),
        ],
        out_shape=[
            jax.ShapeDtypeStruct((1, 1), jnp.float32),
            jax.ShapeDtypeStruct((G, NSUB, RS), jnp.float32),
        ],
    )(E_flat, c3, w2, bias, mask01)
    return loss_sum, mask


def kernel(input_ids, embedding_table, c, W_weight, W_bias):
    Bn, Lp = input_ids.shape
    T = Bn * Lp
    NPH = 4
    TP = T // NPH
    ids_flat = input_ids.reshape(T).astype(jnp.int32)
    gathered = [_sc_gather(ids_flat[k * TP:(k + 1) * TP], embedding_table, c)
                for k in range(NPH)]
    results = [_tc_attention(E, cs, W_weight, W_bias, Lp)
               for E, cs in gathered]
    loss = sum(r[0][0, 0] for r in results) / jnp.float32(T)
    mask = jnp.concatenate([r[1].reshape(TP) for r in results])
    selection_mask = mask.reshape(Bn, Lp) > 0.5
    return loss, selection_mask


# speedup vs baseline: 1.0574x; 1.0574x over previous
import functools
import math

import jax
import jax.numpy as jnp
from jax import lax
from jax.experimental import pallas as pl
from jax.experimental.pallas import tpu as pltpu
from jax.experimental.pallas import tpu_sc as plsc


def _sc_gather(ids_flat, table, c):
    T = ids_flat.shape[0]
    V, Hd = table.shape
    NC, NS = 2, 16
    NW = NC * NS
    t_per_w = T // NW
    CH = 128
    n_ch = t_per_w // CH

    mesh = plsc.VectorSubcoreMesh(core_axis_name="c", subcore_axis_name="s")

    @functools.partial(
        pl.kernel,
        out_type=(
            jax.ShapeDtypeStruct((T, Hd), jnp.float32),
            jax.ShapeDtypeStruct((T,), jnp.float32),
        ),
        mesh=mesh,
        scratch_types=[
            pltpu.VMEM((t_per_w,), jnp.int32),
            pltpu.VMEM((CH, Hd), jnp.float32),
            pltpu.VMEM((CH, Hd), jnp.float32),
            pltpu.VMEM((CH, Hd), jnp.float32),
            pltpu.VMEM((CH, Hd), jnp.float32),
            pltpu.VMEM((t_per_w,), jnp.float32),
            pltpu.SemaphoreType.DMA,
            pltpu.SemaphoreType.DMA,
            pltpu.SemaphoreType.DMA,
        ],
    )
    def sc_kernel(ids_hbm, table_hbm, c_hbm, outE_hbm, outc_hbm,
                  idx_v, b0, b1, b2, b3, cbuf, semG, semW, semC):
        wid = lax.axis_index("s") * NC + lax.axis_index("c")
        base = wid * t_per_w
        pltpu.sync_copy(ids_hbm.at[pl.ds(base, t_per_w)], idx_v)

        ccp = [pltpu.async_copy(c_hbm.at[idx_v.at[pl.ds(i * CH, CH)]],
                                cbuf.at[pl.ds(i * CH, CH)], semC)
               for i in range(n_ch)]

        bufs = (b0, b1, b2, b3)

        def gather(i):
            return pltpu.async_copy(table_hbm.at[idx_v.at[pl.ds(i * CH, CH)]],
                                    bufs[i % 4], semG)

        gcp = [gather(0), gather(1)]
        wcp = []
        for i in range(n_ch):
            if i >= 2:
                wcp[i - 2].wait()
            if i + 2 < n_ch:
                gcp.append(gather(i + 2))
            gcp[i].wait()
            wcp.append(pltpu.async_copy(
                bufs[i % 4], outE_hbm.at[pl.ds(base + i * CH, CH)], semW))
        wcp[n_ch - 2].wait()
        wcp[n_ch - 1].wait()
        for cp in ccp:
            cp.wait()
        pltpu.sync_copy(cbuf, outc_hbm.at[pl.ds(base, t_per_w)])

    return sc_kernel(ids_flat, table, c)


def _tc_body(e_ref, c_ref, w_ref, b_ref, m_ref, loss_ref, mask_ref,
             pre, nsub, rs):
    x = e_ref[...]
    xb = (x * pre).astype(jnp.bfloat16)
    m01 = m_ref[...]
    w2 = w_ref[...]
    ones = jnp.ones((1, rs), dtype=jnp.bfloat16)
    bias = b_ref[0]
    parts = []
    for s in range(nsub):
        xs = xb[s * rs:(s + 1) * rs, :]
        lhs = jnp.concatenate([xs, w2], axis=0)
        gp = lax.dot_general(lhs, xs, (((1,), (1,)), ((), ())),
                             preferred_element_type=jnp.float32)
        g = gp[:rs, :]
        p = gp[rs:rs + 1, :]
        em = jnp.exp2(g) * m01
        em_b = em.astype(jnp.bfloat16)
        denom = jnp.sum(em, axis=0, keepdims=True)
        pp = (p / denom).astype(jnp.bfloat16)
        z = jnp.dot(pp, em_b, preferred_element_type=jnp.float32) + bias
        lq = -jnp.log(1.0 + jnp.exp(-z))
        cb = c_ref[0, s, :].reshape(1, rs)
        parts.append(jnp.sum(cb * (jnp.log(cb) - lq)))
        mask_ref[0, s, :] = jnp.where(cb >= 0.5, 1.0, 0.0).reshape(rs)
    @pl.when(pl.program_id(0) == 0)
    def _():
        loss_ref[0, 0] = 0.0
    loss_ref[0, 0] += sum(parts)


def _tc_attention(E_flat, c_flat, W, bias, Lp, PB=8, NSUB=64):
    T, Hd = E_flat.shape
    RS = PB * Lp
    R = RS * NSUB
    G = T // R
    c3 = c_flat.reshape(G, NSUB, RS)
    blk = jnp.arange(RS, dtype=jnp.int32) // Lp
    mask01 = (blk[:, None] == blk[None, :]).astype(jnp.float32)
    pre = math.sqrt(math.log2(math.e) / math.sqrt(Hd))
    w2 = jnp.zeros((8, Hd), jnp.float32).at[0].set(W.reshape(Hd) / pre)
    w2 = w2.astype(jnp.bfloat16)
    grid = (G,)
    loss_sum, mask = pl.pallas_call(
        functools.partial(_tc_body, pre=pre, nsub=NSUB, rs=RS),
        grid=grid,
        in_specs=[
            pl.BlockSpec((R, Hd), lambda i: (i, 0)),
            pl.BlockSpec((1, NSUB, RS), lambda i: (i, 0, 0)),
            pl.BlockSpec((8, Hd), lambda i: (0, 0)),
            pl.BlockSpec(memory_space=pltpu.SMEM),
            pl.BlockSpec((RS, RS), lambda i: (0, 0)),
        ],
        out_specs=[
            pl.BlockSpec((1, 1), lambda i: (0, 0), memory_space=pltpu.SMEM),
            pl.BlockSpec((1, NSUB, RS), lambda i: (i, 0, 0)),
        ],
        out_shape=[
            jax.ShapeDtypeStruct((1, 1), jnp.float32),
            jax.ShapeDtypeStruct((G, NSUB, RS), jnp.float32),
        ],
    )(E_flat, c3, w2, bias, mask01)
    return loss_sum, mask


def kernel(input_ids, embedding_table, c, W_weight, W_bias):
    Bn, Lp = input_ids.shape
    T = Bn * Lp
    NPH = 2
    TP = T // NPH
    ids_flat = input_ids.reshape(T).astype(jnp.int32)
    gathered = [_sc_gather(ids_flat[k * TP:(k + 1) * TP], embedding_table, c)
                for k in range(NPH)]
    results = [_tc_attention(E, cs, W_weight, W_bias, Lp)
               for E, cs in gathered]
    loss = sum(r[0][0, 0] for r in results) / jnp.float32(T)
    mask = jnp.concatenate([r[1].reshape(TP) for r in results])
    selection_mask = mask.reshape(Bn, Lp) > 0.5
    return loss, selection_mask
